# baseline (speedup 1.0000x reference)
import functools

import numpy as np
import jax
import jax.numpy as jnp
from jax import lax
from jax.experimental import pallas as pl
from jax.experimental.pallas import tpu as pltpu

_B, _N, _DIN, _DOUT = 4, 1024, 256, 128
_MMAX_C = _N * (_N + 1) // 2
_MMAX_A = _N * (_N - 1) // 2


@functools.lru_cache(maxsize=None)
def _edit_consts():
    with jax.ensure_compile_time_eval():
        key = jax.random.key(42)
        k0, k1, k2 = jax.random.split(key, 3)
        noise = np.asarray(
            jax.random.normal(k0, (_B, _N, _DIN), dtype=jnp.float32)) * 0.01

        def perm_tables(k, mmax):
            k, sub = jax.random.split(k)
            bits1 = np.asarray(jax.random.bits(sub, (mmax,), jnp.uint32))
            k, sub = jax.random.split(k)
            bits2 = np.asarray(jax.random.bits(sub, (mmax,), jnp.uint32))
            order1 = np.argsort(bits1, kind="stable").astype(np.int32)
            order2 = np.argsort(bits2, kind="stable").astype(np.int32)
            inv1 = np.empty_like(order1)
            inv1[order1] = np.arange(mmax, dtype=np.int32)
            inv2 = np.empty_like(order2)
            inv2[order2] = np.arange(mmax, dtype=np.int32)
            return order1, inv1, order2, inv2

        return noise, perm_tables(k1, _MMAX_C), perm_tables(k2, _MMAX_A)


def _edit_sel(cand, m, n_sel, tables):
    order1, inv1, order2, inv2 = (jnp.asarray(t) for t in tables)
    mmax = order1.shape[0]
    c1 = jnp.cumsum((order1 < m).astype(jnp.int32))
    rank1 = jnp.clip(c1[inv1] - 1, 0, mmax - 1)
    c2 = jnp.cumsum((order2 < m).astype(jnp.int32))
    jflag = (c2 <= n_sel) & (order2 < m)
    jmask = jflag[inv2]
    pm_two = jmask[rank1]
    pm_one = rank1 < n_sel
    parange = jnp.arange(mmax)
    pmask = jnp.where(m <= 1, parange < n_sel,
                      jnp.where(m <= 1625, pm_one, pm_two)) & (parange < m)
    cf = cand.reshape(-1)
    cfi = cf.astype(jnp.int32)
    ordx = jnp.clip(jnp.cumsum(cfi) - cfi, 0, mmax - 1)
    return (cf & pmask[ordx]).reshape(cand.shape)


def _edited_adjacencies(adj):
    cand_c = jnp.triu(adj) == 1
    m_c = jnp.sum(cand_c.astype(jnp.int32))
    n_sel_c = jnp.minimum(jnp.maximum(1, m_c // 2), m_c)

    triu1 = jnp.triu(jnp.ones((_N, _N), dtype=bool), 1)
    cand_a = (adj == 0) & triu1
    m_a = jnp.sum(cand_a.astype(jnp.int32))
    num_ones = jnp.sum((adj == 1).astype(jnp.int32))
    k = 15 - num_ones
    n_sel_a = jnp.where(k >= 0, jnp.minimum(k, m_a), jnp.maximum(0, m_a + k))

    _, tc, ta = _edit_consts()
    sel_c = _edit_sel(cand_c, m_c, n_sel_c, tc)
    sel_a = _edit_sel(cand_a, m_a, n_sel_a, ta)
    selor_c = sel_c | sel_c.T
    selor_a = sel_a | sel_a.T
    ind = (adj != 0).astype(jnp.float32)
    ind_cut = jnp.where(selor_c, 0.0, ind)
    ind_add = jnp.where(selor_a, 1.0, ind)
    return ind_cut, ind_add


def _pair_mats():
    r = lax.broadcasted_iota(jnp.int32, (8, 4), 0)
    c = lax.broadcasted_iota(jnp.int32, (8, 4), 1)
    p4 = (r // 2 == c).astype(jnp.float32)
    return p4, p4.T


def _dense_body(adj_ref, indc_ref, inda_ref, nodef_ref, noise_ref, init_ref,
                w4_ref, w0_ref, ipwt_ref, ipb_ref, opwt_ref, opb_ref,
                wml_ref, bml_ref,
                z_ref, mu_ref, lv_ref):
    f32 = jnp.float32
    adj = adj_ref[...]
    ind = (adj != 0.0).astype(f32)
    ind_c = indc_ref[...]
    ind_a = inda_ref[...]

    def dinv_of(m):
        deg = jnp.sum(m, axis=0) + jnp.sum(m, axis=1) + 1.0
        return lax.rsqrt(deg)

    dv = dinv_of(ind)
    dv_c = dinv_of(ind_c)
    dv_a = dinv_of(ind_a)

    def prop(m, dinv, u):
        ud = u * dinv[:, None]
        r = (jnp.dot(m, ud, preferred_element_type=f32)
             + lax.dot_general(m, ud, (((0,), (0,)), ((), ())),
                               preferred_element_type=f32)
             + ud)
        return r * dinv[:, None]

    w4 = w4_ref[...]
    w0 = w0_ref[...]

    u_ind_cols = []
    u_c_cols = []
    u_a_cols = []
    for b in range(_B):
        xb = nodef_ref[b]
        u = jnp.dot(xb, w4, preferred_element_type=f32)
        u_v0 = jnp.dot(xb + noise_ref[b], w0, preferred_element_type=f32)
        u_ind_cols.append(u[:, 0:8])
        u_ind_cols.append(u_v0)
        u_c_cols.append(u[:, 8:16])
        u_a_cols.append(u[:, 16:24])
    p_ind = prop(ind, dv, jnp.concatenate(u_ind_cols, axis=1))
    p_c = prop(ind_c, dv_c, jnp.concatenate(u_c_cols, axis=1))
    p_a = prop(ind_a, dv_a, jnp.concatenate(u_a_cols, axis=1))

    p4, p4t = _pair_mats()
    ipwt = ipwt_ref[...]
    ipb = ipb_ref[...]
    opwt = opwt_ref[...]
    opb = opb_ref[...]
    inv_s2 = 1.0 / np.sqrt(2.0)

    fused_cols = []
    for b in range(_B):
        base = p_ind[:, 16 * b: 16 * b + 8]
        v0 = p_ind[:, 16 * b + 8: 16 * b + 16]
        v1 = p_c[:, 8 * b: 8 * b + 8]
        v2 = p_a[:, 8 * b: 8 * b + 8]
        qp = jnp.dot(base, ipwt[:, 0:8], preferred_element_type=f32) + ipb[0, 0:8]
        scs = []
        vps = []
        for v in (v0, v1, v2):
            kp = jnp.dot(v, ipwt[:, 8:16], preferred_element_type=f32) + ipb[0, 8:16]
            vp = jnp.dot(v, ipwt[:, 16:24], preferred_element_type=f32) + ipb[0, 16:24]
            scs.append(jnp.dot(qp * kp, p4, preferred_element_type=f32) * inv_s2)
            vps.append(vp)
        mx = jnp.maximum(jnp.maximum(scs[0], scs[1]), scs[2])
        es = [jnp.exp(s - mx) for s in scs]
        den = es[0] + es[1] + es[2]
        o = sum(jnp.dot(e / den, p4t, preferred_element_type=f32) * vp
                for e, vp in zip(es, vps))
        fused_cols.append(jnp.dot(o, opwt, preferred_element_type=f32) + opb[0])
    p_f = prop(ind, dv, jnp.concatenate(fused_cols, axis=1))

    wml = wml_ref[...]
    bml = bml_ref[...]
    for b in range(_B):
        ml = jnp.dot(p_f[:, 8 * b: 8 * b + 8], wml,
                     preferred_element_type=f32) + bml[0]
        mu = ml[:, 0:_DOUT]
        lv = ml[:, _DOUT:]
        mu_ref[b] = mu
        lv_ref[b] = lv
        z_ref[b] = mu + jnp.exp(lv) * init_ref[b]


def kernel(nodef, adj, init_dist, W_base, b_base, W_v0, b_v0, W_v1, b_v1,
           W_v2, b_v2, in_proj_W, in_proj_b, out_proj_W, out_proj_b,
           W_mu, b_mu, W_lv, b_lv):
    noise, _, _ = _edit_consts()
    ind_cut, ind_add = _edited_adjacencies(adj)

    w4 = jnp.concatenate([W_base, W_v1, W_v2], axis=1)
    ipwt = in_proj_W.T
    opwt = out_proj_W.T
    wml = jnp.concatenate([W_mu, W_lv], axis=1)
    bml = jnp.concatenate([b_mu, b_lv]).reshape(1, 2 * _DOUT)

    out_shape = [jax.ShapeDtypeStruct((_B, _N, _DOUT), jnp.float32)] * 3
    z, mu, lv = pl.pallas_call(
        _dense_body,
        out_shape=out_shape,
    )(adj, ind_cut, ind_add, nodef, jnp.asarray(noise), init_dist,
      w4, W_v0, ipwt, in_proj_b.reshape(1, 24), opwt, out_proj_b.reshape(1, 8),
      wml, bml)
    return (z, mu, lv)


# speedup vs baseline: 20.0660x; 20.0660x over previous
import functools

import numpy as np
import jax
import jax.numpy as jnp
from jax import lax
from jax.experimental import pallas as pl
from jax.experimental.pallas import tpu as pltpu
from jax.experimental.pallas import tpu_sc as plsc

_B, _N, _DIN, _DOUT = 4, 1024, 256, 128
_MMAX_C = _N * (_N + 1) // 2
_MMAX_A = _N * (_N - 1) // 2
_NS = 16
_RC_C = _MMAX_C // _NS
_RC_A = _MMAX_A // _NS
_SUB_C, _NSUB_C = 3280, 10
_SUB_A, _NSUB_A = 2976, 11
_PMW = 16640
_STR = _PMW // _NS
_IVW = 32816


@functools.lru_cache(maxsize=None)
def _edit_consts():
    with jax.set_mesh(None), jax.ensure_compile_time_eval():
        key = jax.random.key(42)
        k0, k1, k2 = jax.random.split(key, 3)
        noise = np.asarray(
            jax.random.normal(k0, (_B, _N, _DIN), dtype=jnp.float32)) * 0.01

        def perm_tables(k, mmax):
            k, sub = jax.random.split(k)
            bits1 = np.asarray(jax.random.bits(sub, (mmax,), jnp.uint32))
            k, sub = jax.random.split(k)
            bits2 = np.asarray(jax.random.bits(sub, (mmax,), jnp.uint32))
            order1 = np.argsort(bits1, kind="stable").astype(np.int32)
            order2 = np.argsort(bits2, kind="stable").astype(np.int32)
            inv2 = np.empty_like(order2)
            inv2[order2] = np.arange(mmax, dtype=np.int32)
            return order1, order2, inv2

        return noise, perm_tables(k1, _MMAX_C), perm_tables(k2, _MMAX_A)


_edit_consts()


def _lanes():
    return lax.iota(jnp.int32, 16)


def _bcast(cond):
    return jnp.broadcast_to(cond, (16,))


def _spi(x):
    return jnp.full((16,), x, jnp.int32)


def _sc_body(adj_ref, o1c_ref, o2c_ref, i2c_ref, o1a_ref, o2a_ref, i2a_ref,
             selc_ref, sela_ref,
             v_ord1, v_ord2, v_inv2, v_pmC, v_pmA, v_rbC, v_rbA,
             v_cntC, v_cntA, v_cntO, v_adjrow, v_rowC, v_rowA, v_tmp, v_tot,
             s_pm, s_comb, s_cnt, s_tot, s_misc):
    i32 = jnp.int32
    s = lax.axis_index("s")
    core = lax.axis_index("c")
    lanes = _lanes()

    def row_stats(ri, _):
        i = s * 64 + ri
        pltpu.sync_copy(adj_ref.at[i], v_adjrow)

        def chunk(k, accs):
            aC, aA, aO = accs
            a = v_adjrow[pl.ds(k * 16, 16)]
            j = _spi(k * 16) + lanes
            iv = _spi(i)
            one = a == 1.0
            aC = aC + ((j >= iv) & one).astype(i32)
            aA = aA + ((j > iv) & (a == 0.0)).astype(i32)
            aO = aO + one.astype(i32)
            return aC, aA, aO

        z = jnp.zeros((16,), i32)
        aC, aA, aO = lax.fori_loop(0, 64, chunk, (z, z, z))
        ii = jnp.full((16,), i, i32)
        lane0 = lanes == 0
        plsc.store_scatter(v_cntC, [ii], jnp.full((16,), jnp.sum(aC), i32),
                           mask=lane0)
        plsc.store_scatter(v_cntA, [ii], jnp.full((16,), jnp.sum(aA), i32),
                           mask=lane0)
        plsc.store_scatter(v_cntO, [ii], jnp.full((16,), jnp.sum(aO), i32),
                           mask=lane0)
        return 0

    lax.fori_loop(0, 64, row_stats, 0)
    pltpu.sync_copy(v_cntC.at[pl.ds(s * 64, 64)],
                    s_cnt.at[pl.ds(s * 64, 64)])
    pltpu.sync_copy(v_cntA.at[pl.ds(s * 64, 64)],
                    s_cnt.at[pl.ds(_N + s * 64, 64)])
    pltpu.sync_copy(v_cntO.at[pl.ds(s * 64, 64)],
                    s_cnt.at[pl.ds(2 * _N + s * 64, 64)])
    plsc.subcore_barrier()
    pltpu.sync_copy(s_cnt.at[pl.ds(0, _N)], v_cntC)
    pltpu.sync_copy(s_cnt.at[pl.ds(_N, _N)], v_cntA)
    pltpu.sync_copy(s_cnt.at[pl.ds(2 * _N, _N)], v_cntO)

    def scan_rows(cnt_ref, rb_ref):
        def body(k, carry):
            v = cnt_ref[pl.ds(k * 16, 16)]
            cs = plsc.cumsum(v)
            rb_ref[pl.ds(k * 16, 16)] = cs - v + _spi(carry)
            return carry + jnp.sum(v)
        return lax.fori_loop(0, 64, body, jnp.int32(0))

    m_c = scan_rows(v_cntC, v_rbC)
    m_a = scan_rows(v_cntA, v_rbA)

    def sum_rows(cnt_ref):
        def body(k, acc):
            return acc + jnp.sum(cnt_ref[pl.ds(k * 16, 16)])
        return lax.fori_loop(0, 64, body, jnp.int32(0))

    num_ones = sum_rows(v_cntO)
    n_sel_c = jnp.minimum(jnp.maximum(1, m_c // 2), m_c)
    ka = 15 - num_ones
    n_sel_a = jnp.where(ka >= 0, jnp.minimum(ka, m_a),
                        jnp.maximum(0, m_a + ka))

    def run_edit(o1_ref, o2_ref, i2_ref, v_pmX, rc, sub, nsub, mmax, m, n_sel):
        nv1 = rc // 16
        base = s * rc
        pltpu.sync_copy(o1_ref.at[pl.ds(base, rc)], v_ord1.at[pl.ds(0, rc)])

        mv = _spi(m)

        def tot1_body(k, acc):
            v = v_ord1[pl.ds(k * 16, 16)]
            return acc + (v < mv).astype(i32)

        tot1 = jnp.sum(lax.fori_loop(0, nv1, tot1_body, jnp.zeros((16,), i32)))

        def tot2_sub(q, acc):
            pltpu.sync_copy(o2_ref.at[pl.ds(base + q * sub, sub)],
                            v_ord2.at[pl.ds(0, sub)])

            def inner(k, a2):
                v = v_ord2[pl.ds(k * 16, 16)]
                return a2 + (v < mv).astype(i32)

            return lax.fori_loop(0, sub // 16, inner, acc)

        tot2 = jnp.sum(lax.fori_loop(0, nsub, tot2_sub, jnp.zeros((16,), i32)))
        zv = jnp.zeros((16,), i32)
        v_tmp[...] = jnp.where(lanes == 0, _spi(tot1),
                               jnp.where(lanes == 1, _spi(tot2), zv))
        pltpu.sync_copy(v_tmp.at[pl.ds(0, 8)], s_tot.at[pl.ds(s * 8, 8)])

        @pl.when(s == 0)
        def _():
            v_tmp[...] = jnp.where(lanes == 0, jnp.full((16,), -1, i32), zv)
            pltpu.sync_copy(v_tmp.at[pl.ds(0, 8)], s_misc)

        plsc.subcore_barrier()

        pltpu.sync_copy(s_tot, v_tot)

        sv = _spi(s)

        def off_body(k, offs):
            o1, o2 = offs
            p = _spi(k * 16) + lanes
            t = jnp.right_shift(p, 3)
            rem = p & 7
            v = v_tot[pl.ds(k * 16, 16)]
            o1 = o1 + jnp.sum(jnp.where((rem == 0) & (t < sv), v, zv))
            o2 = o2 + jnp.sum(jnp.where((rem == 1) & (t < sv), v, zv))
            return o1, o2

        off1, off2 = lax.fori_loop(0, _NS * 8 // 16, off_body,
                                   (jnp.int32(0), jnp.int32(0)))

        def search_sub(q, carry):
            pos, cnt = carry
            pltpu.sync_copy(o2_ref.at[pl.ds(base + q * sub, sub)],
                            v_ord2.at[pl.ds(0, sub)])

            def inner(k, c2):
                pos2, cnt2 = c2
                v = v_ord2[pl.ds(k * 16, 16)]
                p = (v < mv).astype(i32)
                cs = plsc.cumsum(p) + _spi(cnt2)
                match = (p == 1) & (cs == _spi(n_sel))
                lane = jnp.sum(jnp.where(match, lanes, zv))
                hit = jnp.sum(match.astype(i32)) > 0
                cand = base + q * sub + k * 16 + lane
                pos2 = jnp.where((pos2 < 0) & hit, cand, pos2)
                return pos2, cnt2 + jnp.sum(p)

            return lax.fori_loop(0, sub // 16, inner, (pos, cnt))

        pos, _ = lax.fori_loop(0, nsub, search_sub, (jnp.int32(-1), off2))

        @pl.when((pos >= 0) & (n_sel > 0))
        def _():
            v_tmp[...] = jnp.where(lanes == 0, _spi(pos), zv)
            pltpu.sync_copy(v_tmp.at[pl.ds(0, 8)], s_misc)

        plsc.subcore_barrier()
        pltpu.sync_copy(s_misc, v_tmp.at[pl.ds(0, 8)])
        r2s = jnp.sum(jnp.where(lanes == 0, v_tmp[...], zv))

        def zero_body(k, _):
            v_pmX[pl.ds(k * 16, 16)] = jnp.zeros((16,), i32)
            return 0

        lax.fori_loop(0, _PMW // 16, zero_body, 0)
        astart = jnp.maximum(0, jnp.minimum(off1 & ~7, mmax - _IVW))
        astart = pl.multiple_of(astart, 8)
        delta = off1 - astart
        pltpu.sync_copy(i2_ref.at[pl.ds(astart, _IVW)], v_inv2)

        nsv = _spi(n_sel)
        r2sv = _spi(r2s)
        deltav = _spi(delta)
        off1v = _spi(off1)
        onev = jnp.full((16,), 1, i32)
        m_le1 = jnp.full((16,), m <= 1, jnp.bool_)
        m_le1625 = jnp.full((16,), m <= 1625, jnp.bool_)

        def p2_body(k, carry):
            v = v_ord1[pl.ds(k * 16, 16)]
            predb = v < mv
            p = predb.astype(i32)
            cs = plsc.cumsum(p) + _spi(carry)
            idx = jnp.maximum(cs - onev + deltav, zv)
            iv = plsc.load_gather(v_inv2, [idx])
            bit_two = predb & (iv <= r2sv)
            bit_one = predb & (cs - onev + off1v < nsv)
            bit_id = v < nsv
            bitb = jnp.where(m_le1, bit_id,
                             jnp.where(m_le1625, bit_one, bit_two))
            word = jnp.right_shift(v, 5)
            bval = jnp.where(bitb, jnp.left_shift(onev, v & 31), zv)
            plsc.addupdate_scatter(v_pmX, [word], bval)
            return carry + jnp.sum(p)

        lax.fori_loop(0, nv1, p2_body, jnp.int32(0))

        pltpu.sync_copy(v_pmX, s_pm.at[pl.ds(s * _PMW, _PMW)])
        plsc.subcore_barrier()
        for t in range(_NS):
            pltpu.sync_copy(s_pm.at[pl.ds(t * _PMW + s * _STR, _STR)],
                            v_inv2.at[pl.ds(t * _STR, _STR)])

        def or_body(k, _):
            acc = jnp.zeros((16,), i32)
            for t in range(_NS):
                acc = acc | v_inv2[pl.ds(t * _STR + k * 16, 16)]
            v_pmX[pl.ds(k * 16, 16)] = acc
            return 0

        lax.fori_loop(0, _STR // 16, or_body, 0)
        pltpu.sync_copy(v_pmX.at[pl.ds(0, _STR)],
                        s_comb.at[pl.ds(s * _STR, _STR)])
        plsc.subcore_barrier()
        pltpu.sync_copy(s_comb, v_pmX)
        plsc.subcore_barrier()

    run_edit(o1c_ref, o2c_ref, i2c_ref, v_pmC,
             _RC_C, _SUB_C, _NSUB_C, _MMAX_C, m_c, n_sel_c)
    run_edit(o1a_ref, o2a_ref, i2a_ref, v_pmA,
             _RC_A, _SUB_A, _NSUB_A, _MMAX_A, m_a, n_sel_a)

    def cell_row(ri, _):
        i = core * 512 + s * 32 + ri
        pltpu.sync_copy(adj_ref.at[i], v_adjrow)
        ii = jnp.full((16,), i, i32)
        rbC = plsc.load_gather(v_rbC, [ii])
        rbA = plsc.load_gather(v_rbA, [ii])

        def chunk(k, carry):
            cC, cA = carry
            a = v_adjrow[pl.ds(k * 16, 16)]
            j = _spi(k * 16) + lanes
            iv = jnp.full((16,), i, i32)
            candC = (j >= iv) & (a == 1.0)
            candA = (j > iv) & (a == 0.0)
            ci = candC.astype(i32)
            ai = candA.astype(i32)
            pC = cC + plsc.cumsum(ci) - ci
            pA = cA + plsc.cumsum(ai) - ai
            wC = plsc.load_gather(v_pmC, [jnp.right_shift(pC, 5)])
            wA = plsc.load_gather(v_pmA, [jnp.right_shift(pA, 5)])
            bC = jnp.right_shift(wC, pC & 31) & 1
            bA = jnp.right_shift(wA, pA & 31) & 1
            v_rowC[pl.ds(k * 16, 16)] = (candC & (bC == 1)).astype(jnp.float32)
            v_rowA[pl.ds(k * 16, 16)] = (candA & (bA == 1)).astype(jnp.float32)
            return cC + _spi(jnp.sum(ci)), cA + _spi(jnp.sum(ai))

        lax.fori_loop(0, 64, chunk, (rbC, rbA))
        pltpu.sync_copy(v_rowC, selc_ref.at[i])
        pltpu.sync_copy(v_rowA, sela_ref.at[i])
        return 0

    lax.fori_loop(0, 32, cell_row, 0)


def _sc_select(adj):
    _, tc, ta = _edit_consts()
    o1c, o2c, i2c = (jnp.asarray(t) for t in tc)
    o1a, o2a, i2a = (jnp.asarray(t) for t in ta)
    mesh = plsc.VectorSubcoreMesh(core_axis_name="c", subcore_axis_name="s")
    f = pl.kernel(
        _sc_body,
        out_type=[jax.ShapeDtypeStruct((_N, _N), jnp.float32),
                  jax.ShapeDtypeStruct((_N, _N), jnp.float32)],
        mesh=mesh,
        compiler_params=pltpu.CompilerParams(needs_layout_passes=False),
        scratch_types=[
            pltpu.VMEM((_RC_C,), jnp.int32),
            pltpu.VMEM((_SUB_C,), jnp.int32),
            pltpu.VMEM((_IVW,), jnp.int32),
            pltpu.VMEM((_PMW,), jnp.int32),
            pltpu.VMEM((_PMW,), jnp.int32),
            pltpu.VMEM((_N,), jnp.int32),
            pltpu.VMEM((_N,), jnp.int32),
            pltpu.VMEM((_N,), jnp.int32),
            pltpu.VMEM((_N,), jnp.int32),
            pltpu.VMEM((_N,), jnp.int32),
            pltpu.VMEM((_N,), jnp.float32),
            pltpu.VMEM((_N,), jnp.float32),
            pltpu.VMEM((_N,), jnp.float32),
            pltpu.VMEM((16,), jnp.int32),
            pltpu.VMEM((_NS * 8,), jnp.int32),
            pltpu.VMEM_SHARED((_NS * _PMW,), jnp.int32),
            pltpu.VMEM_SHARED((_PMW,), jnp.int32),
            pltpu.VMEM_SHARED((3 * _N,), jnp.int32),
            pltpu.VMEM_SHARED((_NS * 8,), jnp.int32),
            pltpu.VMEM_SHARED((8,), jnp.int32),
        ],
    )
    return f(adj, o1c, o2c, i2c, o1a, o2a, i2a)


def _pair_mats():
    r = lax.broadcasted_iota(jnp.int32, (8, 4), 0)
    c = lax.broadcasted_iota(jnp.int32, (8, 4), 1)
    p4 = (r // 2 == c).astype(jnp.float32)
    return p4, p4.T


def _dense_body(adj_ref, selc_ref, sela_ref, eye_ref, nodef_ref, noise_ref,
                init_ref, w4_ref, w0_ref, ipwt_ref, ipb_ref, opwt_ref,
                opb_ref, wml_ref, bml_ref,
                z_ref, mu_ref, lv_ref):
    f32 = jnp.float32
    adj = adj_ref[...]
    eye = eye_ref[...]
    ind = (adj != 0.0).astype(f32)

    def transpose(m):
        return lax.dot_general(m.astype(jnp.bfloat16), eye,
                               (((0,), (0,)), ((), ())),
                               preferred_element_type=f32)

    selc = selc_ref[...]
    sela = sela_ref[...]
    sc_or = selc + transpose(selc)
    sa_or = sela + transpose(sela)
    ind_c = jnp.where(sc_or > 0.0, 0.0, ind)
    ind_a = jnp.where(sa_or > 0.0, 1.0, ind)

    def dinv_of(m):
        deg = jnp.sum(m, axis=0) + jnp.sum(m, axis=1) + 1.0
        return lax.rsqrt(deg)

    dv = dinv_of(ind)
    dv_c = dinv_of(ind_c)
    dv_a = dinv_of(ind_a)

    def prop(m, dinv, u):
        ud = u * dinv[:, None]
        r = (jnp.dot(m, ud, preferred_element_type=f32)
             + lax.dot_general(m, ud, (((0,), (0,)), ((), ())),
                               preferred_element_type=f32)
             + ud)
        return r * dinv[:, None]

    w4 = w4_ref[...]
    w0 = w0_ref[...]

    u_ind_cols = []
    u_c_cols = []
    u_a_cols = []
    for b in range(_B):
        xb = nodef_ref[b]
        u = jnp.dot(xb, w4, preferred_element_type=f32)
        u_v0 = jnp.dot(xb + noise_ref[b], w0, preferred_element_type=f32)
        u_ind_cols.append(u[:, 0:8])
        u_ind_cols.append(u_v0)
        u_c_cols.append(u[:, 8:16])
        u_a_cols.append(u[:, 16:24])
    p_ind = prop(ind, dv, jnp.concatenate(u_ind_cols, axis=1))
    p_c = prop(ind_c, dv_c, jnp.concatenate(u_c_cols, axis=1))
    p_a = prop(ind_a, dv_a, jnp.concatenate(u_a_cols, axis=1))

    p4, p4t = _pair_mats()
    ipwt = ipwt_ref[...]
    ipb = ipb_ref[...]
    opwt = opwt_ref[...]
    opb = opb_ref[...]
    inv_s2 = 1.0 / np.sqrt(2.0)

    fused_cols = []
    for b in range(_B):
        base = p_ind[:, 16 * b: 16 * b + 8]
        v0 = p_ind[:, 16 * b + 8: 16 * b + 16]
        v1 = p_c[:, 8 * b: 8 * b + 8]
        v2 = p_a[:, 8 * b: 8 * b + 8]
        qp = jnp.dot(base, ipwt[:, 0:8], preferred_element_type=f32) + ipb[0, 0:8]
        scs = []
        vps = []
        for v in (v0, v1, v2):
            kp = jnp.dot(v, ipwt[:, 8:16], preferred_element_type=f32) + ipb[0, 8:16]
            vp = jnp.dot(v, ipwt[:, 16:24], preferred_element_type=f32) + ipb[0, 16:24]
            scs.append(jnp.dot(qp * kp, p4, preferred_element_type=f32) * inv_s2)
            vps.append(vp)
        mx = jnp.maximum(jnp.maximum(scs[0], scs[1]), scs[2])
        es = [jnp.exp(sx - mx) for sx in scs]
        den = es[0] + es[1] + es[2]
        o = sum(jnp.dot(e / den, p4t, preferred_element_type=f32) * vp
                for e, vp in zip(es, vps))
        fused_cols.append(jnp.dot(o, opwt, preferred_element_type=f32) + opb[0])
    p_f = prop(ind, dv, jnp.concatenate(fused_cols, axis=1))

    wml = wml_ref[...]
    bml = bml_ref[...]
    for b in range(_B):
        ml = jnp.dot(p_f[:, 8 * b: 8 * b + 8], wml,
                     preferred_element_type=f32) + bml[0]
        mu = ml[:, 0:_DOUT]
        lv = ml[:, _DOUT:]
        mu_ref[b] = mu
        lv_ref[b] = lv
        z_ref[b] = mu + jnp.exp(lv) * init_ref[b]


def kernel(nodef, adj, init_dist, W_base, b_base, W_v0, b_v0, W_v1, b_v1,
           W_v2, b_v2, in_proj_W, in_proj_b, out_proj_W, out_proj_b,
           W_mu, b_mu, W_lv, b_lv):
    noise, _, _ = _edit_consts()
    selc, sela = _sc_select(adj)

    w4 = jnp.concatenate([W_base, W_v1, W_v2], axis=1)
    ipwt = in_proj_W.T
    opwt = out_proj_W.T
    wml = jnp.concatenate([W_mu, W_lv], axis=1)
    bml = jnp.concatenate([b_mu, b_lv]).reshape(1, 2 * _DOUT)
    eye = jnp.asarray(np.eye(_N, dtype=np.float32), dtype=jnp.bfloat16)

    out_shape = [jax.ShapeDtypeStruct((_B, _N, _DOUT), jnp.float32)] * 3
    z, mu, lv = pl.pallas_call(
        _dense_body,
        out_shape=out_shape,
    )(adj, selc, sela, eye, nodef, jnp.asarray(noise), init_dist,
      w4, W_v0, ipwt, in_proj_b.reshape(1, 24), opwt, out_proj_b.reshape(1, 8),
      wml, bml)
    return (z, mu, lv)
